# baseline (speedup 1.0000x reference)
import functools

import jax
import jax.numpy as jnp
from jax import lax
from jax.experimental import pallas as pl
from jax.experimental.pallas import tpu as pltpu
from jax.experimental.pallas import tpu_sc as plsc

_NUM_EMB = 100000
_SIZE_EMB = 16
_NC = 2
_NS = 16
_NW = _NC * _NS
_L = 16

_CH = 512
_GSUB = 128
_K = _CH // _GSUB


def _make_kernel(n_pos, hw):
    per_w = n_pos // _NW
    n_chunks = per_w // _CH
    scale = float(_NUM_EMB - 1)

    mesh = plsc.VectorSubcoreMesh(core_axis_name="c", subcore_axis_name="s")

    @functools.partial(
        pl.kernel,
        out_type=jax.ShapeDtypeStruct((n_pos // hw * _SIZE_EMB, hw), jnp.float32),
        mesh=mesh,
        compiler_params=pltpu.CompilerParams(
            needs_layout_passes=False, use_tc_tiling_on_sc=False),
        scratch_types=[
            pltpu.VMEM((_CH,), jnp.float32),
            pltpu.VMEM((_CH,), jnp.float32),
            pltpu.VMEM((_K, _GSUB), jnp.int32),
            pltpu.VMEM((_K, _GSUB), jnp.int32),
            pltpu.VMEM((_CH, _SIZE_EMB), jnp.float32),
            pltpu.VMEM((_CH, _SIZE_EMB), jnp.float32),
            pltpu.VMEM((_SIZE_EMB, _CH), jnp.float32),
            pltpu.VMEM_SHARED((_NUM_EMB, _SIZE_EMB), jnp.float32),
            pltpu.SemaphoreType.DMA,
            pltpu.SemaphoreType.DMA,
            pltpu.SemaphoreType.DMA,
            pltpu.SemaphoreType.DMA,
            pltpu.SemaphoreType.DMA,
        ],
    )
    def k(feat_hbm, table_hbm, out_hbm,
          feat0, feat1, idx0, idx1, rows0, rows1, outt, table_sp,
          fsem0, fsem1, gsem0, gsem1, osem):
        wid = lax.axis_index("s") * _NC + lax.axis_index("c")
        base = wid * per_w
        n_w_per_b = hw // per_w
        b = wid // n_w_per_b
        col0 = (wid % n_w_per_b) * per_w
        brow = b * _SIZE_EMB

        bufs = ((feat0, idx0, rows0, fsem0, gsem0),
                (feat1, idx1, rows1, fsem1, gsem1))

        def fire_feat(c, feat_v, fsem):
            pltpu.async_copy(feat_hbm.at[pl.ds(base + c * _CH, _CH)],
                             feat_v, fsem)

        def wait_feat(feat_v, fsem):
            pltpu.make_async_copy(feat_hbm.at[pl.ds(0, _CH)],
                                  feat_v, fsem).wait()

        def compute_idx(feat_v, idx_v):
            def idx_body(j, _):
                for g in range(_GSUB // _L):
                    v = feat_v[pl.ds(j * _GSUB + g * _L, _L)]
                    idx_v[j, pl.ds(g * _L, _L)] = (v * scale).astype(jnp.int32)
                return ()
            lax.fori_loop(0, _K, idx_body, (), unroll=False)

        def fire_gathers(idx_v, rows_v, gsem):
            for j in range(_K):
                pltpu.async_copy(table_sp.at[idx_v.at[j]],
                                 rows_v.at[pl.ds(j * _GSUB, _GSUB)], gsem)

        def wait_gathers(rows_v, gsem):
            pltpu.make_async_copy(table_hbm.at[pl.ds(0, _CH)],
                                  rows_v, gsem).wait()

        def transpose(rows_v, outt_v):
            lane = lax.iota(jnp.int32, _L)
            def tr_body(i, _):
                row_idx = i * _L + lane
                for c in range(_SIZE_EMB):
                    col_idx = jnp.full((_L,), c, jnp.int32)
                    outt_v[c, pl.ds(i * _L, _L)] = plsc.load_gather(
                        rows_v, [row_idx, col_idx])
                return ()
            lax.fori_loop(0, _CH // _L, tr_body, (), unroll=False)

        def fire_out(c, outt_v, osem):
            pltpu.async_copy(
                outt_v,
                out_hbm.at[pl.ds(brow, _SIZE_EMB),
                           pl.ds(col0 + c * _CH, _CH)],
                osem)

        def wait_out(outt_v, osem):
            pltpu.make_async_copy(
                outt_v,
                out_hbm.at[pl.ds(brow, _SIZE_EMB), pl.ds(col0, _CH)],
                osem).wait()

        def chunk_step(c, cur, nxt):
            feat_c, idx_c, rows_c, fsem_c, gsem_c = cur
            feat_n, idx_n, rows_n, fsem_n, gsem_n = nxt
            @pl.when(c + 1 < n_chunks)
            def _():
                wait_feat(feat_n, fsem_n)
                compute_idx(feat_n, idx_n)
                fire_gathers(idx_n, rows_n, gsem_n)
                @pl.when(c + 2 < n_chunks)
                def _():
                    fire_feat(c + 2, feat_c, fsem_c)
            wait_gathers(rows_c, gsem_c)
            @pl.when(c >= 1)
            def _():
                wait_out(outt, osem)
            transpose(rows_c, outt)
            fire_out(c, outt, osem)

        @pl.when(lax.axis_index("s") == 0)
        def _():
            pltpu.sync_copy(table_hbm, table_sp)
        plsc.subcore_barrier()

        fire_feat(0, feat0, fsem0)
        wait_feat(feat0, fsem0)
        compute_idx(feat0, idx0)
        fire_gathers(idx0, rows0, gsem0)
        fire_feat(1, feat1, fsem1)

        def pair_body(kk, _):
            chunk_step(2 * kk, bufs[0], bufs[1])
            chunk_step(2 * kk + 1, bufs[1], bufs[0])
            return ()
        lax.fori_loop(0, n_chunks // 2, pair_body, (), unroll=False)

        wait_out(outt, osem)

    return k


def kernel(feature, table):
    B, C, H, W = feature.shape
    hw = H * W
    n_pos = B * C * H * W
    feat_flat = feature.reshape(n_pos)
    out2d = _make_kernel(n_pos, hw)(feat_flat, table)
    return out2d.reshape(B, _SIZE_EMB, H, W)


# speedup vs baseline: 1.5902x; 1.5902x over previous
import functools

import jax
import jax.numpy as jnp
from jax import lax
from jax.experimental import pallas as pl
from jax.experimental.pallas import tpu as pltpu
from jax.experimental.pallas import tpu_sc as plsc

_NUM_EMB = 100000
_SIZE_EMB = 16
_NC = 2
_NS = 16
_NW = _NC * _NS
_L = 16

_CH = 512
_GSUB = 128
_K = _CH // _GSUB


def _make_kernel(n_pos, hw):
    per_w = n_pos // _NW
    n_chunks = per_w // _CH
    scale = float(_NUM_EMB - 1)

    mesh = plsc.VectorSubcoreMesh(core_axis_name="c", subcore_axis_name="s")

    @functools.partial(
        pl.kernel,
        out_type=jax.ShapeDtypeStruct((n_pos // hw * _SIZE_EMB, hw), jnp.float32),
        mesh=mesh,
        compiler_params=pltpu.CompilerParams(
            needs_layout_passes=False, use_tc_tiling_on_sc=False),
        scratch_types=[
            pltpu.VMEM((_CH,), jnp.float32),
            pltpu.VMEM((_CH,), jnp.float32),
            pltpu.VMEM((_K, _GSUB), jnp.int32),
            pltpu.VMEM((_K, _GSUB), jnp.int32),
            pltpu.VMEM((_CH, _SIZE_EMB), jnp.float32),
            pltpu.VMEM((_CH, _SIZE_EMB), jnp.float32),
            pltpu.VMEM((_SIZE_EMB, _CH + 1), jnp.float32),
            pltpu.VMEM_SHARED((_NUM_EMB, _SIZE_EMB), jnp.float32),
            pltpu.SemaphoreType.DMA,
            pltpu.SemaphoreType.DMA,
            pltpu.SemaphoreType.DMA,
            pltpu.SemaphoreType.DMA,
            pltpu.SemaphoreType.DMA,
        ],
    )
    def k(feat_hbm, table_hbm, out_hbm,
          feat0, feat1, idx0, idx1, rows0, rows1, outt, table_sp,
          fsem0, fsem1, gsem0, gsem1, osem):
        wid = lax.axis_index("s") * _NC + lax.axis_index("c")
        base = wid * per_w
        n_w_per_b = hw // per_w
        b = wid // n_w_per_b
        col0 = (wid % n_w_per_b) * per_w
        brow = b * _SIZE_EMB

        bufs = ((feat0, idx0, rows0, fsem0, gsem0),
                (feat1, idx1, rows1, fsem1, gsem1))

        def fire_feat(c, feat_v, fsem):
            pltpu.async_copy(feat_hbm.at[pl.ds(base + c * _CH, _CH)],
                             feat_v, fsem)

        def wait_feat(feat_v, fsem):
            pltpu.make_async_copy(feat_hbm.at[pl.ds(0, _CH)],
                                  feat_v, fsem).wait()

        def compute_idx(feat_v, idx_v):
            def idx_body(j, _):
                for g in range(_GSUB // _L):
                    v = feat_v[pl.ds(j * _GSUB + g * _L, _L)]
                    idx_v[j, pl.ds(g * _L, _L)] = (v * scale).astype(jnp.int32)
                return ()
            lax.fori_loop(0, _K, idx_body, (), unroll=False)

        def fire_gathers(idx_v, rows_v, gsem):
            for j in range(_K):
                pltpu.async_copy(table_sp.at[idx_v.at[j]],
                                 rows_v.at[pl.ds(j * _GSUB, _GSUB)], gsem)

        def wait_gathers(rows_v, gsem):
            pltpu.make_async_copy(table_hbm.at[pl.ds(0, _CH)],
                                  rows_v, gsem).wait()

        def transpose(rows_v, outt_v):
            lane = lax.iota(jnp.int32, _L)
            def tr_body(i, _):
                p0 = i * 8
                for u in range(8):
                    v = rows_v[p0 + u]
                    plsc.store_scatter(outt_v, [lane, jnp.full((_L,), 0, jnp.int32) + (p0 + u)], v)
                return ()
            lax.fori_loop(0, _CH // 8, tr_body, (), unroll=False)

        def fire_out(c, outt_v, osem):
            pltpu.async_copy(
                outt_v.at[:, pl.ds(0, _CH)],
                out_hbm.at[pl.ds(brow, _SIZE_EMB),
                           pl.ds(col0 + c * _CH, _CH)],
                osem)

        def wait_out(outt_v, osem):
            pltpu.make_async_copy(
                outt_v.at[:, pl.ds(0, _CH)],
                out_hbm.at[pl.ds(brow, _SIZE_EMB), pl.ds(col0, _CH)],
                osem).wait()

        def chunk_step(c, cur, nxt):
            feat_c, idx_c, rows_c, fsem_c, gsem_c = cur
            feat_n, idx_n, rows_n, fsem_n, gsem_n = nxt
            @pl.when(c + 1 < n_chunks)
            def _():
                wait_feat(feat_n, fsem_n)
                compute_idx(feat_n, idx_n)
                fire_gathers(idx_n, rows_n, gsem_n)
                @pl.when(c + 2 < n_chunks)
                def _():
                    fire_feat(c + 2, feat_c, fsem_c)
            wait_gathers(rows_c, gsem_c)
            @pl.when(c >= 1)
            def _():
                wait_out(outt, osem)
            transpose(rows_c, outt)
            fire_out(c, outt, osem)

        @pl.when(lax.axis_index("s") == 0)
        def _():
            pltpu.sync_copy(table_hbm, table_sp)
        plsc.subcore_barrier()

        fire_feat(0, feat0, fsem0)
        wait_feat(feat0, fsem0)
        compute_idx(feat0, idx0)
        fire_gathers(idx0, rows0, gsem0)
        fire_feat(1, feat1, fsem1)

        def pair_body(kk, _):
            chunk_step(2 * kk, bufs[0], bufs[1])
            chunk_step(2 * kk + 1, bufs[1], bufs[0])
            return ()
        lax.fori_loop(0, n_chunks // 2, pair_body, (), unroll=False)

        wait_out(outt, osem)

    return k


def kernel(feature, table):
    B, C, H, W = feature.shape
    hw = H * W
    n_pos = B * C * H * W
    feat_flat = feature.reshape(n_pos)
    out2d = _make_kernel(n_pos, hw)(feat_flat, table)
    return out2d.reshape(B, _SIZE_EMB, H, W)
